# Initial kernel scaffold; baseline (speedup 1.0000x reference)
import jax
import jax.numpy as jnp
from jax.experimental import pallas as pl


def kernel(input_ids, word_table, pos_table, gamma, beta):
    raise NotImplementedError("write your pallas kernel here")


# speedup vs baseline: 2.1315x; 2.1315x over previous
import functools

import jax
import jax.numpy as jnp
from jax import lax
from jax.experimental import pallas as pl
from jax.experimental.pallas import tpu as pltpu
from jax.experimental.pallas import tpu_sc as plsc

LANES = 16
EPS = 1e-12
MAGIC = 0x5F3759DF


def _splat_sum(x):
    iota = lax.iota(jnp.int32, LANES)
    for k in (1, 2, 4, 8):
        perm = jnp.bitwise_xor(iota, jnp.int32(k))
        x = x + x.at[perm].get(mode="promise_in_bounds")
    return x


def _rsqrt(x):
    i = lax.bitcast_convert_type(x, jnp.int32)
    i = jnp.int32(MAGIC) - lax.shift_right_logical(i, 1)
    y = lax.bitcast_convert_type(i, jnp.float32)
    half = x * jnp.float32(0.5)
    for _ in range(3):
        y = y * (jnp.float32(1.5) - half * y * y)
    return y


@functools.partial(jax.jit, static_argnames=("batch", "seq", "hidden", "chunk"))
def _run(input_ids, word_table, pos_table, gamma, beta, *, batch, seq, hidden, chunk):
    nc, ns = 2, 16
    nw = nc * ns
    seq_per_w = batch // nw
    n_chunks = seq // chunk
    nh = hidden // LANES
    mesh = plsc.VectorSubcoreMesh(
        core_axis_name="c", subcore_axis_name="s", num_cores=nc, num_subcores=ns
    )

    @functools.partial(
        pl.kernel,
        out_type=jax.ShapeDtypeStruct((batch, seq, hidden), jnp.float32),
        mesh=mesh,
        scratch_types=[
            pltpu.VMEM((seq, hidden), jnp.float32),
            pltpu.VMEM((hidden,), jnp.float32),
            pltpu.VMEM((hidden,), jnp.float32),
            pltpu.VMEM((chunk,), jnp.int32),
            pltpu.VMEM((chunk, hidden), jnp.float32),
            pltpu.SemaphoreType.DMA,
        ],
    )
    def k(ids_hbm, word_hbm, pos_hbm, gamma_hbm, beta_hbm, out_hbm,
          pos_v, gamma_v, beta_v, idx_v, rows_v, sem):
        wid = lax.axis_index("s") * nc + lax.axis_index("c")
        pltpu.sync_copy(pos_hbm, pos_v)
        pltpu.sync_copy(gamma_hbm, gamma_v)
        pltpu.sync_copy(beta_hbm, beta_v)

        inv_h = jnp.float32(1.0 / hidden)

        def chunk_body(cid, _):
            s_loc = cid // n_chunks
            c = cid % n_chunks
            seq_id = wid * seq_per_w + s_loc
            base = c * chunk
            pltpu.sync_copy(ids_hbm.at[seq_id, pl.ds(base, chunk)], idx_v)
            pltpu.async_copy(word_hbm.at[idx_v], rows_v, sem).wait()

            def tok_body(t, _):
                vs = [
                    rows_v[t, pl.ds(LANES * j, LANES)]
                    + pos_v[base + t, pl.ds(LANES * j, LANES)]
                    for j in range(nh)
                ]
                acc = vs[0]
                acc2 = vs[0] * vs[0]
                for j in range(1, nh):
                    acc = acc + vs[j]
                    acc2 = acc2 + vs[j] * vs[j]
                u_v = _splat_sum(acc) * inv_h
                m2_v = _splat_sum(acc2) * inv_h
                var_v = m2_v - u_v * u_v
                inv = _rsqrt(var_v + jnp.float32(EPS))
                for j in range(nh):
                    g = gamma_v[pl.ds(LANES * j, LANES)] * inv
                    b = beta_v[pl.ds(LANES * j, LANES)]
                    rows_v[t, pl.ds(LANES * j, LANES)] = (vs[j] - u_v) * g + b
                return 0

            lax.fori_loop(0, chunk, tok_body, 0)
            pltpu.sync_copy(rows_v, out_hbm.at[seq_id, pl.ds(base, chunk)])
            return 0

        lax.fori_loop(0, seq_per_w * n_chunks, chunk_body, 0)

    return k(input_ids, word_table, pos_table, gamma, beta)


def kernel(input_ids, word_table, pos_table, gamma, beta):
    batch, seq = input_ids.shape
    hidden = word_table.shape[1]
    return _run(
        input_ids.astype(jnp.int32), word_table, pos_table, gamma, beta,
        batch=batch, seq=seq, hidden=hidden, chunk=128,
    )


# speedup vs baseline: 3.2305x; 1.5156x over previous
import functools

import jax
import jax.numpy as jnp
from jax import lax
from jax.experimental import pallas as pl
from jax.experimental.pallas import tpu as pltpu
from jax.experimental.pallas import tpu_sc as plsc

LANES = 16
EPS = 1e-12
MAGIC = 0x5F3759DF


def _splat_sum(x):
    iota = lax.iota(jnp.int32, LANES)
    for k in (1, 2, 4, 8):
        perm = jnp.bitwise_xor(iota, jnp.int32(k))
        x = x + x.at[perm].get(mode="promise_in_bounds")
    return x


def _rsqrt(x):
    i = lax.bitcast_convert_type(x, jnp.int32)
    i = jnp.int32(MAGIC) - lax.shift_right_logical(i, 1)
    y = lax.bitcast_convert_type(i, jnp.float32)
    half = x * jnp.float32(0.5)
    for _ in range(3):
        y = y * (jnp.float32(1.5) - half * y * y)
    return y


@functools.partial(jax.jit, static_argnames=("batch", "seq", "hidden", "chunk"))
def _run(input_ids, word_table, pos_table, gamma, beta, *, batch, seq, hidden, chunk):
    nc, ns = 2, 16
    nw = nc * ns
    seq_per_w = batch // nw
    n_chunks = seq // chunk
    nh = hidden // LANES
    mesh = plsc.VectorSubcoreMesh(
        core_axis_name="c", subcore_axis_name="s", num_cores=nc, num_subcores=ns
    )

    @functools.partial(
        pl.kernel,
        out_type=jax.ShapeDtypeStruct((batch, seq, hidden), jnp.float32),
        mesh=mesh,
        scratch_types=[
            pltpu.VMEM((seq, hidden), jnp.float32),
            pltpu.VMEM((hidden,), jnp.float32),
            pltpu.VMEM((hidden,), jnp.float32),
            pltpu.VMEM((chunk,), jnp.int32),
            pltpu.VMEM((chunk, hidden), jnp.float32),
            pltpu.SemaphoreType.DMA,
        ],
    )
    def k(ids_hbm, word_hbm, pos_hbm, gamma_hbm, beta_hbm, out_hbm,
          pos_v, gamma_v, beta_v, idx_v, rows_v, sem):
        wid = lax.axis_index("s") * nc + lax.axis_index("c")
        pltpu.sync_copy(pos_hbm, pos_v)
        pltpu.sync_copy(gamma_hbm, gamma_v)
        pltpu.sync_copy(beta_hbm, beta_v)

        inv_h = jnp.float32(1.0 / hidden)

        def chunk_body(cid, _):
            s_loc = cid // n_chunks
            c = cid % n_chunks
            seq_id = wid * seq_per_w + s_loc
            base = c * chunk
            pltpu.sync_copy(ids_hbm.at[seq_id, pl.ds(base, chunk)], idx_v)
            pltpu.async_copy(word_hbm.at[idx_v], rows_v, sem).wait()

            @plsc.parallel_loop(0, chunk, step=1, unroll=4)
            def tok_body(t):
                vs = [
                    rows_v[t, pl.ds(LANES * j, LANES)]
                    + pos_v[base + t, pl.ds(LANES * j, LANES)]
                    for j in range(nh)
                ]
                acc = vs[0]
                acc2 = vs[0] * vs[0]
                for j in range(1, nh):
                    acc = acc + vs[j]
                    acc2 = acc2 + vs[j] * vs[j]
                u_v = _splat_sum(acc) * inv_h
                m2_v = _splat_sum(acc2) * inv_h
                var_v = m2_v - u_v * u_v
                inv = _rsqrt(var_v + jnp.float32(EPS))
                for j in range(nh):
                    g = gamma_v[pl.ds(LANES * j, LANES)] * inv
                    b = beta_v[pl.ds(LANES * j, LANES)]
                    rows_v[t, pl.ds(LANES * j, LANES)] = (vs[j] - u_v) * g + b

            pltpu.sync_copy(rows_v, out_hbm.at[seq_id, pl.ds(base, chunk)])
            return 0

        lax.fori_loop(0, seq_per_w * n_chunks, chunk_body, 0)

    return k(input_ids, word_table, pos_table, gamma, beta)


def kernel(input_ids, word_table, pos_table, gamma, beta):
    batch, seq = input_ids.shape
    hidden = word_table.shape[1]
    return _run(
        input_ids.astype(jnp.int32), word_table, pos_table, gamma, beta,
        batch=batch, seq=seq, hidden=hidden, chunk=128,
    )


# speedup vs baseline: 4.7887x; 1.4823x over previous
import functools

import jax
import jax.numpy as jnp
from jax import lax
from jax.experimental import pallas as pl
from jax.experimental.pallas import tpu as pltpu
from jax.experimental.pallas import tpu_sc as plsc

LANES = 16
EPS = 1e-12
MAGIC = 0x5F3759DF


def _splat_sum(x):
    iota = lax.iota(jnp.int32, LANES)
    for k in (1, 2, 4, 8):
        perm = jnp.bitwise_xor(iota, jnp.int32(k))
        x = x + x.at[perm].get(mode="promise_in_bounds")
    return x


def _rsqrt(x):
    i = lax.bitcast_convert_type(x, jnp.int32)
    i = jnp.int32(MAGIC) - lax.shift_right_logical(i, 1)
    y = lax.bitcast_convert_type(i, jnp.float32)
    half = x * jnp.float32(0.5)
    for _ in range(3):
        y = y * (jnp.float32(1.5) - half * y * y)
    return y


@functools.partial(jax.jit, static_argnames=("batch", "seq", "hidden", "chunk"))
def _run(input_ids, word_table, pos_table, gamma, beta, *, batch, seq, hidden, chunk):
    nc, ns = 2, 16
    nw = nc * ns
    seq_per_w = batch // nw
    n_chunks = seq // chunk
    nh = hidden // LANES
    mesh = plsc.VectorSubcoreMesh(
        core_axis_name="c", subcore_axis_name="s", num_cores=nc, num_subcores=ns
    )

    @functools.partial(
        pl.kernel,
        out_type=jax.ShapeDtypeStruct((batch, seq, hidden), jnp.float32),
        mesh=mesh,
        scratch_types=[
            pltpu.VMEM((seq, hidden), jnp.float32),
            pltpu.VMEM((hidden,), jnp.float32),
            pltpu.VMEM((hidden,), jnp.float32),
            pltpu.VMEM((chunk,), jnp.int32),
            pltpu.VMEM((chunk, hidden), jnp.float32),
            pltpu.SemaphoreType.DMA,
        ],
    )
    def k(ids_hbm, word_hbm, pos_hbm, gamma_hbm, beta_hbm, out_hbm,
          pos_v, gamma_v, beta_v, idx_v, rows_v, sem):
        wid = lax.axis_index("s") * nc + lax.axis_index("c")
        pltpu.sync_copy(pos_hbm, pos_v)
        pltpu.sync_copy(gamma_hbm, gamma_v)
        pltpu.sync_copy(beta_hbm, beta_v)

        inv_h = jnp.float32(1.0 / hidden)

        def chunk_body(cid, _):
            s_loc = cid // n_chunks
            c = cid % n_chunks
            seq_id = wid * seq_per_w + s_loc
            base = c * chunk
            pltpu.sync_copy(ids_hbm.at[seq_id, pl.ds(base, chunk)], idx_v)
            pltpu.async_copy(word_hbm.at[idx_v], rows_v, sem).wait()

            @plsc.parallel_loop(0, chunk, step=1, unroll=8)
            def tok_body(t):
                vs = [
                    rows_v[t, pl.ds(LANES * j, LANES)]
                    + pos_v[base + t, pl.ds(LANES * j, LANES)]
                    for j in range(nh)
                ]
                acc = vs[0]
                acc2 = vs[0] * vs[0]
                for j in range(1, nh):
                    acc = acc + vs[j]
                    acc2 = acc2 + vs[j] * vs[j]
                u_v = _splat_sum(acc) * inv_h
                m2_v = _splat_sum(acc2) * inv_h
                var_v = m2_v - u_v * u_v
                inv = _rsqrt(var_v + jnp.float32(EPS))
                for j in range(nh):
                    g = gamma_v[pl.ds(LANES * j, LANES)] * inv
                    b = beta_v[pl.ds(LANES * j, LANES)]
                    rows_v[t, pl.ds(LANES * j, LANES)] = (vs[j] - u_v) * g + b

            pltpu.sync_copy(rows_v, out_hbm.at[seq_id, pl.ds(base, chunk)])
            return 0

        lax.fori_loop(0, seq_per_w * n_chunks, chunk_body, 0)

    return k(input_ids, word_table, pos_table, gamma, beta)


def kernel(input_ids, word_table, pos_table, gamma, beta):
    batch, seq = input_ids.shape
    hidden = word_table.shape[1]
    return _run(
        input_ids.astype(jnp.int32), word_table, pos_table, gamma, beta,
        batch=batch, seq=seq, hidden=hidden, chunk=128,
    )
